# Initial kernel scaffold; baseline (speedup 1.0000x reference)
import jax
import jax.numpy as jnp
from jax.experimental import pallas as pl


def kernel(nodes, edges, batch, W1, b1, W2, b2, Wp, bp, W3, b3, W4, b4, W5, b5):
    raise NotImplementedError("write your pallas kernel here")


# speedup vs baseline: 5.6405x; 5.6405x over previous
import jax
import jax.numpy as jnp
from jax import lax
from jax.experimental import pallas as pl
from jax.experimental.pallas import tpu as pltpu
from jax.experimental.pallas import tpu_sc as plsc

N = 4096
E = 131072
D = 128
K = 102
EPS = 1e-15

NC, NS = 2, 16
TRASH = N
ACC_ROWS = N + NS
EPT = E // (NC * NS)
CH = 128
NCH = EPT // CH

def _mesh():
  return plsc.VectorSubcoreMesh(core_axis_name="c", subcore_axis_name="s",
                                num_cores=NC, num_subcores=NS)


def _edge_agg_body(y, gidx, sidx, zrows, out, acc, gbuf, sbuf, rows, sem):
  c = lax.axis_index("c")
  s = lax.axis_index("s")
  pltpu.sync_copy(zrows, acc.at[pl.ds(s * 257, 257)])
  plsc.subcore_barrier()
  base = (c * NS + s) * EPT

  def chunk(k, carry):
    off = base + k * CH
    pltpu.sync_copy(gidx.at[pl.ds(off, CH)], gbuf)
    pltpu.sync_copy(sidx.at[pl.ds(off, CH)], sbuf)
    pltpu.async_copy(y.at[gbuf], rows, sem).wait()
    pltpu.sync_copy(rows, acc.at[sbuf], add=True)
    return carry

  lax.fori_loop(0, NCH, chunk, 0)
  plsc.subcore_barrier()
  pltpu.sync_copy(acc.at[pl.ds(s * 256, 256)], out.at[c, pl.ds(s * 256, 256)])


def _edge_agg(y, gidx, sidx, zrows):
  return pl.kernel(
      _edge_agg_body,
      out_type=jax.ShapeDtypeStruct((NC, N, D), jnp.float32),
      mesh=_mesh(),
      scratch_types=[
          pltpu.VMEM_SHARED((ACC_ROWS, D), jnp.float32),
          pltpu.VMEM((CH,), jnp.int32),
          pltpu.VMEM((CH,), jnp.int32),
          pltpu.VMEM((CH, D), jnp.float32),
          pltpu.SemaphoreType.DMA,
      ],
  )(y, gidx, sidx, zrows)


DEG_ROWS = 4352


def _degree_body(dstidx, sidxd, out, acca, accb, ibuf, jbuf, vbuf, zq, sem):
  c = lax.axis_index("c")
  s = lax.axis_index("s")

  def fz(i, carry):
    zq[pl.ds(i * 16, 16)] = jnp.zeros((16,), jnp.float32)
    return carry

  lax.fori_loop(0, 272 // 16, fz, 0)

  def fo(i, carry):
    vbuf[pl.ds(i * 16, 16)] = jnp.full((16,), 1.0, jnp.float32)
    return carry

  lax.fori_loop(0, CH // 16, fo, 0)
  pltpu.sync_copy(zq, acca.at[pl.ds(s * 272, 272)])
  pltpu.sync_copy(zq, accb.at[pl.ds(s * 272, 272)])
  plsc.subcore_barrier()
  base = (c * NS + s) * EPT

  def chunk(k, carry):
    off = base + k * CH
    pltpu.sync_copy(dstidx.at[pl.ds(off, CH)], ibuf)
    pltpu.sync_copy(sidxd.at[pl.ds(off, CH)], jbuf)
    pltpu.sync_copy(vbuf, acca.at[ibuf], add=True)
    pltpu.sync_copy(vbuf, accb.at[jbuf], add=True)
    return carry

  lax.fori_loop(0, NCH, chunk, 0)
  plsc.subcore_barrier()
  pltpu.sync_copy(acca.at[pl.ds(s * 256, 256)],
                  out.at[c * 2, pl.ds(s * 256, 256)])
  pltpu.sync_copy(accb.at[pl.ds(s * 256, 256)],
                  out.at[c * 2 + 1, pl.ds(s * 256, 256)])


def _degrees(dstidx, sidxd):
  return pl.kernel(
      _degree_body,
      out_type=jax.ShapeDtypeStruct((2 * NC, N), jnp.float32),
      mesh=_mesh(),
      scratch_types=[
          pltpu.VMEM_SHARED((DEG_ROWS,), jnp.float32),
          pltpu.VMEM_SHARED((DEG_ROWS,), jnp.float32),
          pltpu.VMEM((CH,), jnp.int32),
          pltpu.VMEM((CH,), jnp.int32),
          pltpu.VMEM((CH,), jnp.float32),
          pltpu.VMEM((272,), jnp.float32),
          pltpu.SemaphoreType.DMA,
      ],
  )(dstidx, sidxd)


RNG = 1 << 20
RWIN = 8
EPT2 = E // NS
ZCH = 16384


def _dup_body(ids, zflat, out, table, ids_v, xbuf, vbuf, rbuf, abuf, sem):
  c = lax.axis_index("c")
  s = lax.axis_index("s")
  pltpu.sync_copy(ids.at[pl.ds(s * EPT2, EPT2)], ids_v)
  spread = lax.iota(jnp.int32, 16) * 64

  def window(r, tot):
    base_id = (c * RWIN + r) * RNG
    for j in range(RNG // NS // ZCH):
      pltpu.sync_copy(zflat, table.at[pl.ds(s * (RNG // NS) + j * ZCH, ZCH)])
    plsc.subcore_barrier()

    def scat(k, carry):
      def sub(j, carry2):
        v = ids_v[pl.ds(k * CH + j * 16, 16)]
        local = v - base_id
        m = (local >= 0) & (local < RNG)
        xbuf[pl.ds(j * 16, 16)] = jnp.where(m, local, spread + j * 1024)
        vbuf[pl.ds(j * 16, 16)] = jnp.where(m, 1.0, 0.0)
        return carry2

      lax.fori_loop(0, CH // 16, sub, 0)
      pltpu.sync_copy(vbuf, table.at[xbuf], add=True)
      return carry

    lax.fori_loop(0, EPT2 // CH, scat, 0)
    plsc.subcore_barrier()

    def gath(k, acc):
      def sub(j, carry2):
        v = ids_v[pl.ds(k * CH + j * 16, 16)]
        local = v - base_id
        m = (local >= 0) & (local < RNG)
        xbuf[pl.ds(j * 16, 16)] = jnp.where(m, local, spread + j * 1024)
        return carry2

      lax.fori_loop(0, CH // 16, sub, 0)
      pltpu.sync_copy(table.at[xbuf], rbuf)

      def sub2(j, acc2):
        v = ids_v[pl.ds(k * CH + j * 16, 16)]
        local = v - base_id
        m = (local >= 0) & (local < RNG)
        return acc2 + jnp.where(m, rbuf[pl.ds(j * 16, 16)], 0.0)

      return lax.fori_loop(0, CH // 16, sub2, acc)

    tot = lax.fori_loop(0, EPT2 // CH, gath, tot)
    plsc.subcore_barrier()
    return tot

  tot = lax.fori_loop(0, RWIN, window, jnp.zeros((16,), jnp.float32))
  abuf[...] = tot
  pltpu.sync_copy(abuf, out.at[c, s])


def _dupcount(ids, zflat):
  return pl.kernel(
      _dup_body,
      out_type=jax.ShapeDtypeStruct((NC, NS, 16), jnp.float32),
      mesh=_mesh(),
      scratch_types=[
          pltpu.VMEM_SHARED((RNG,), jnp.float32),
          pltpu.VMEM((EPT2,), jnp.int32),
          pltpu.VMEM((CH,), jnp.int32),
          pltpu.VMEM((CH,), jnp.float32),
          pltpu.VMEM((CH,), jnp.float32),
          pltpu.VMEM((16,), jnp.float32),
          pltpu.SemaphoreType.DMA,
      ],
  )(ids, zflat)


R = 256
G = N // R
_f32 = jnp.float32


def _dot(a, b):
  return jnp.dot(a, b, preferred_element_type=_f32)


def _prep_body(src_ref, dst_ref, sidx_ref, ids_ref):
  s = src_ref[...]
  d = dst_ref[...]
  sidx_ref[...] = jnp.where(s == d, TRASH, s)
  ids_ref[...] = s * N + d


def _prep(srcm, dstm):
  return pl.pallas_call(
      _prep_body,
      out_shape=(
          jax.ShapeDtypeStruct((E // D, D), jnp.int32),
          jax.ShapeDtypeStruct((E // D, D), jnp.int32),
      ),
  )(srcm, dstm)


def _dinv_body(degp_ref, a_ref, b_ref):
  dp = degp_ref[...]
  a_ref[...] = lax.rsqrt(1.0 + dp[0] + dp[2])
  b_ref[...] = lax.rsqrt(1.0 + dp[1] + dp[3])


def _dinv(degpm):
  return pl.pallas_call(
      _dinv_body,
      out_shape=(
          jax.ShapeDtypeStruct((N // D, D), _f32),
          jax.ShapeDtypeStruct((N // D, D), _f32),
      ),
  )(degpm)


def _matscale_body(x_ref, w_ref, scale_ref, o_ref):
  o_ref[...] = scale_ref[...] * _dot(x_ref[...], w_ref[...])


def _matscale(x, w, scale):
  return pl.pallas_call(
      _matscale_body,
      grid=(G,),
      in_specs=[
          pl.BlockSpec((R, D), lambda i: (i, 0)),
          pl.BlockSpec((D, D), lambda i: (0, 0)),
          pl.BlockSpec((R, D), lambda i: (i, 0)),
      ],
      out_specs=pl.BlockSpec((R, D), lambda i: (i, 0)),
      out_shape=jax.ShapeDtypeStruct((N, D), _f32),
  )(x, w, scale)


def _convpost_body(p_ref, y_ref, dinv_ref, b_ref, w_ref, o_ref):
  p = p_ref[...]
  dinv = dinv_ref[...]
  h = jnp.tanh(dinv * (p[0] + p[1] + y_ref[...]) + b_ref[...])
  o_ref[...] = dinv * _dot(h, w_ref[...])


def _convpost(p, y, dinv, b, w):
  return pl.pallas_call(
      _convpost_body,
      grid=(G,),
      in_specs=[
          pl.BlockSpec((NC, R, D), lambda i: (0, i, 0)),
          pl.BlockSpec((R, D), lambda i: (i, 0)),
          pl.BlockSpec((R, D), lambda i: (i, 0)),
          pl.BlockSpec((1, D), lambda i: (0, 0)),
          pl.BlockSpec((D, D), lambda i: (0, 0)),
      ],
      out_specs=pl.BlockSpec((R, D), lambda i: (i, 0)),
      out_shape=jax.ShapeDtypeStruct((N, D), _f32),
  )(p, y, dinv, b, w)


def _convpost2_body(p_ref, y_ref, dinv_ref, dinv2_ref, b_ref, w_ref, x_ref,
                    yp_ref):
  p = p_ref[...]
  dinv = dinv_ref[...]
  h = jnp.tanh(dinv * (p[0] + p[1] + y_ref[...]) + b_ref[...])
  x_ref[...] = h
  yp_ref[...] = dinv2_ref[...] * _dot(h, w_ref[...])


def _convpost2(p, y, dinv, dinv2, b, w):
  return pl.pallas_call(
      _convpost2_body,
      grid=(G,),
      in_specs=[
          pl.BlockSpec((NC, R, D), lambda i: (0, i, 0)),
          pl.BlockSpec((R, D), lambda i: (i, 0)),
          pl.BlockSpec((R, D), lambda i: (i, 0)),
          pl.BlockSpec((R, D), lambda i: (i, 0)),
          pl.BlockSpec((1, D), lambda i: (0, 0)),
          pl.BlockSpec((D, D), lambda i: (0, 0)),
      ],
      out_specs=(
          pl.BlockSpec((R, D), lambda i: (i, 0)),
          pl.BlockSpec((R, D), lambda i: (i, 0)),
      ),
      out_shape=(
          jax.ShapeDtypeStruct((N, D), _f32),
          jax.ShapeDtypeStruct((N, D), _f32),
      ),
  )(p, y, dinv, dinv2, b, w)


def _softmax_body(p_ref, y_ref, dinv2_ref, b_ref, s_ref, ent_ref):
  i = pl.program_id(0)
  p = p_ref[...]
  dinv2 = dinv2_ref[...]
  sl = dinv2 * (p[0] + p[1] + y_ref[...]) + b_ref[...]
  col = lax.broadcasted_iota(jnp.int32, (R, D), 1)
  sl = jnp.where(col < K, sl, -1e30)
  m = jnp.max(sl, axis=1, keepdims=True)
  e = jnp.exp(sl - m)
  e = jnp.where(col < K, e, 0.0)
  s = e / jnp.sum(e, axis=1, keepdims=True)
  s_ref[...] = s
  ent = jnp.sum(-s * jnp.log(s + EPS))

  @pl.when(i == 0)
  def _():
    ent_ref[...] = jnp.zeros_like(ent_ref)

  colv = lax.broadcasted_iota(jnp.int32, (1, D), 1)
  ent_ref[...] += jnp.where(colv == 0, ent, 0.0)


def _softmax(p, y, dinv2, b):
  return pl.pallas_call(
      _softmax_body,
      grid=(G,),
      in_specs=[
          pl.BlockSpec((NC, R, D), lambda i: (0, i, 0)),
          pl.BlockSpec((R, D), lambda i: (i, 0)),
          pl.BlockSpec((R, D), lambda i: (i, 0)),
          pl.BlockSpec((1, D), lambda i: (0, 0)),
      ],
      out_specs=(
          pl.BlockSpec((R, D), lambda i: (i, 0)),
          pl.BlockSpec((1, D), lambda i: (0, 0)),
      ),
      out_shape=(
          jax.ShapeDtypeStruct((N, D), _f32),
          jax.ShapeDtypeStruct((1, D), _f32),
      ),
  )(p, y, dinv2, b)


def _contract_body(s_ref, pv_ref, x_ref, adjp_ref, xp_ref, scal_ref, g_acc):
  i = pl.program_id(0)

  @pl.when(i == 0)
  def _():
    adjp_ref[...] = jnp.zeros_like(adjp_ref)
    xp_ref[...] = jnp.zeros_like(xp_ref)
    scal_ref[...] = jnp.zeros_like(scal_ref)
    g_acc[...] = jnp.zeros_like(g_acc)

  s = s_ref[...]
  pv = pv_ref[...]
  v = pv[0] + pv[1]
  dn = (((0,), (0,)), ((), ()))
  adjp_ref[...] += lax.dot_general(s, v, dn, preferred_element_type=_f32)
  xp_ref[...] += lax.dot_general(s, x_ref[...], dn, preferred_element_type=_f32)
  g_acc[...] += lax.dot_general(s, s, dn, preferred_element_type=_f32)
  colv = lax.broadcasted_iota(jnp.int32, (1, D), 1)
  scal_ref[...] += jnp.where(colv == 0, jnp.sum(s * v), 0.0)

  @pl.when(i == G - 1)
  def _():
    g = g_acc[...]
    scal_ref[...] += jnp.where(colv == 1, jnp.sum(g * g), 0.0)


def _contract(s, pv, x):
  return pl.pallas_call(
      _contract_body,
      grid=(G,),
      in_specs=[
          pl.BlockSpec((R, D), lambda i: (i, 0)),
          pl.BlockSpec((NC, R, D), lambda i: (0, i, 0)),
          pl.BlockSpec((R, D), lambda i: (i, 0)),
      ],
      out_specs=(
          pl.BlockSpec((D, D), lambda i: (0, 0)),
          pl.BlockSpec((D, D), lambda i: (0, 0)),
          pl.BlockSpec((1, D), lambda i: (0, 0)),
      ),
      out_shape=(
          jax.ShapeDtypeStruct((D, D), _f32),
          jax.ShapeDtypeStruct((D, D), _f32),
          jax.ShapeDtypeStruct((1, D), _f32),
      ),
      scratch_shapes=[pltpu.VMEM((D, D), _f32)],
  )(s, pv, x)


def _pool_body(s_ref, adjp_ref, xp_ref, w3_ref, dinv_ref, b1_ref, y3_ref):
  s = s_ref[...]
  b1_ref[...] = _dot(s, adjp_ref[...])
  x_out = _dot(s, xp_ref[...])
  y3_ref[...] = dinv_ref[...] * _dot(x_out, w3_ref[...])


def _pool(s, adjp, xp, w3, dinv):
  return pl.pallas_call(
      _pool_body,
      grid=(G,),
      in_specs=[
          pl.BlockSpec((R, D), lambda i: (i, 0)),
          pl.BlockSpec((D, D), lambda i: (0, 0)),
          pl.BlockSpec((D, D), lambda i: (0, 0)),
          pl.BlockSpec((D, D), lambda i: (0, 0)),
          pl.BlockSpec((R, D), lambda i: (i, 0)),
      ],
      out_specs=(
          pl.BlockSpec((R, D), lambda i: (i, 0)),
          pl.BlockSpec((R, D), lambda i: (i, 0)),
      ),
      out_shape=(
          jax.ShapeDtypeStruct((N, D), _f32),
          jax.ShapeDtypeStruct((N, D), _f32),
      ),
  )(s, adjp, xp, w3, dinv)


def _final_body(p_ref, y_ref, dinv_ref, b_ref, o_ref):
  p = p_ref[...]
  o_ref[...] = dinv_ref[...] * (p[0] + p[1] + y_ref[...]) + b_ref[...]


def _final(p, y, dinv, b):
  return pl.pallas_call(
      _final_body,
      grid=(G,),
      in_specs=[
          pl.BlockSpec((NC, R, D), lambda i: (0, i, 0)),
          pl.BlockSpec((R, D), lambda i: (i, 0)),
          pl.BlockSpec((R, D), lambda i: (i, 0)),
          pl.BlockSpec((1, D), lambda i: (0, 0)),
      ],
      out_specs=pl.BlockSpec((R, D), lambda i: (i, 0)),
      out_shape=jax.ShapeDtypeStruct((N, D), _f32),
  )(p, y, dinv, b)


def _adjout_body(b1_ref, s_ref, o_ref):
  dn = (((1,), (1,)), ((), ()))
  o_ref[...] = lax.dot_general(b1_ref[...], s_ref[...], dn,
                               preferred_element_type=_f32)[None]


def _adjout(b1, s):
  return pl.pallas_call(
      _adjout_body,
      grid=(G,),
      in_specs=[
          pl.BlockSpec((R, D), lambda i: (i, 0)),
          pl.BlockSpec((N, D), lambda i: (0, 0)),
      ],
      out_specs=pl.BlockSpec((1, R, N), lambda i: (0, i, 0)),
      out_shape=jax.ShapeDtypeStruct((1, N, N), _f32),
  )(b1, s)


def kernel(nodes, edges, batch, W1, b1, W2, b2, Wp, bp, W3, b3, W4, b4, W5,
           b5):
  del batch
  src = edges[0]
  dst = edges[1]
  sidxm, idsm = _prep(src.reshape(E // D, D), dst.reshape(E // D, D))
  sidxd = sidxm.reshape(E)
  ids = idsm.reshape(E)

  zrows = jnp.zeros((257, D), _f32)
  zflat = jnp.zeros((ZCH,), _f32)

  degp = _degrees(dst, sidxd)
  dupp = _dupcount(ids, zflat)

  wp_pad = jnp.pad(Wp, ((0, 0), (0, D - K)))
  bp_pad = jnp.pad(bp, (0, D - K)).reshape(1, D)
  b1r = b1.reshape(1, D)
  b2r = b2.reshape(1, D)
  b3r = b3.reshape(1, D)
  b4r = b4.reshape(1, D)
  b5r = b5.reshape(1, D)

  dm, d2m = _dinv(degp.reshape(2 * NC, N // D, D))
  dinvc = jnp.broadcast_to(dm.reshape(N)[:, None], (N, D))
  dinv2c = jnp.broadcast_to(d2m.reshape(N)[:, None], (N, D))

  y1 = _matscale(nodes, W1, dinvc)
  p1 = _edge_agg(y1, src, dst, zrows)
  y2 = _convpost(p1, y1, dinvc, b1r, W2)
  p2 = _edge_agg(y2, src, dst, zrows)
  x, yp = _convpost2(p2, y2, dinvc, dinv2c, b2r, wp_pad)
  pp = _edge_agg(yp, dst, sidxd, zrows)
  s_mat, ent_sum = _softmax(pp, yp, dinv2c, bp_pad)
  pv = _edge_agg(s_mat, dst, src, zrows)
  adjp, xp, scal = _contract(s_mat, pv, x)
  b1m, y3 = _pool(s_mat, adjp, xp, W3, dinvc)
  p3 = _edge_agg(y3, src, dst, zrows)
  y4 = _convpost(p3, y3, dinvc, b3r, W4)
  p4 = _edge_agg(y4, src, dst, zrows)
  y5 = _convpost(p4, y4, dinvc, b4r, W5)
  p5 = _edge_agg(y5, src, dst, zrows)
  x2 = _final(p5, y5, dinvc, b5r)
  adj_out = _adjout(b1m, s_mat)

  sumc2 = jnp.sum(dupp)
  cross = scal[0, 0]
  norm_g2 = scal[0, 1]
  link2 = sumc2 - 2.0 * cross + norm_g2
  l1 = 0.1 * jnp.sqrt(link2) / (N * N)
  e1 = 0.1 * ent_sum[0, 0] / N
  return (x2, adj_out, l1, e1)
